# baseline (speedup 1.0000x reference)
import functools

import jax
import jax.numpy as jnp
from jax import lax
from jax.experimental import pallas as pl
from jax.experimental.pallas import tpu as pltpu
from jax.experimental.pallas import tpu_sc as plsc

HIDDEN = 128
NTYPES = 6
NCOMBO = NTYPES ** 3
NC, NS = 2, 16
NW = NC * NS
NBUF = 5


def _build(num_bonds: int, chunk: int):
  bonds_per_w = num_bonds // NW
  assert bonds_per_w * NW == num_bonds
  nchunk = bonds_per_w // chunk
  assert nchunk * chunk == bonds_per_w and chunk % 16 == 0
  assert nchunk % NBUF == 0 and nchunk >= 2 * NBUF
  nlap = nchunk // NBUF

  def body(attr_hbm, w0_hbm, w1_hbm, w2_hbm, out_hbm, *scratch):
    w0_v, w1_v, w2_v, table_v = scratch[:4]
    attr_bufs = scratch[4:4 + NBUF]
    stage_bufs = scratch[4 + NBUF:4 + 2 * NBUF]
    asems = scratch[4 + 2 * NBUF:4 + 3 * NBUF]
    osems = scratch[4 + 3 * NBUF:4 + 4 * NBUF]

    wid = lax.axis_index("s") * NC + lax.axis_index("c")
    base_bond = wid * bonds_per_w

    for s in range(NBUF):
      pltpu.async_copy(
          attr_hbm.at[pl.ds(base_bond + s * chunk, chunk)],
          attr_bufs[s].at[pl.ds(0, chunk)], asems[s])

    pltpu.sync_copy(w0_hbm, w0_v)
    pltpu.sync_copy(w1_hbm, w1_v)
    pltpu.sync_copy(w2_hbm, w2_v)

    def bi0(i0, c0):
      def bi1(i1, c1):
        def bi2(i2, c2):
          base = (i0 * 36 + i1 * 6 + i2) * HIDDEN
          for k in range(HIDDEN // 16):
            o = k * 16
            table_v[pl.ds(base + o, 16)] = (
                w0_v[pl.ds(i0 * HIDDEN + o, 16)]
                + w1_v[pl.ds(i1 * HIDDEN + o, 16)]
                + w2_v[pl.ds(i2 * HIDDEN + o, 16)])
          return c2
        return lax.fori_loop(0, NTYPES, bi2, c1)
      return lax.fori_loop(0, NTYPES, bi1, c0)
    lax.fori_loop(0, NTYPES, bi0, 0)

    def lap_body(lap, carry):
      for s in range(NBUF):
        t = lap * NBUF + s
        cb = base_bond + t * chunk
        attr_v = attr_bufs[s]
        stage_v = stage_bufs[s]

        pltpu.make_async_copy(
            attr_hbm.at[pl.ds(0, chunk)],
            attr_v.at[pl.ds(0, chunk)], asems[s]).wait()

        @pl.when(lap > 0)
        def _():
          pltpu.make_async_copy(
              stage_v, out_hbm.at[pl.ds(0, chunk), :], osems[s]).wait()

        @plsc.parallel_loop(0, chunk // 16, unroll=2)
        def group_body(g):
          av = attr_v[pl.ds(g * 16, 16)]
          basev = ((av >> 8) * 36 + ((av >> 4) & 15) * NTYPES
                   + (av & 15)) * HIDDEN
          for i in range(16):
            b = g * 16 + i
            base = basev[i]
            for k in range(HIDDEN // 16):
              o = k * 16
              stage_v[b, pl.ds(o, 16)] = table_v[pl.ds(base + o, 16)]

        pltpu.async_copy(stage_v, out_hbm.at[pl.ds(cb, chunk), :], osems[s])

        @pl.when(lap < nlap - 1)
        def _():
          pltpu.async_copy(
              attr_hbm.at[pl.ds(cb + NBUF * chunk, chunk)],
              attr_v.at[pl.ds(0, chunk)], asems[s])
      return carry
    lax.fori_loop(0, nlap, lap_body, 0)

    for s in range(NBUF):
      pltpu.make_async_copy(
          stage_bufs[s], out_hbm.at[pl.ds(0, chunk), :], osems[s]).wait()

  mesh = plsc.VectorSubcoreMesh(
      core_axis_name="c", subcore_axis_name="s", num_cores=NC, num_subcores=NS)
  scratch = (
      [pltpu.VMEM((NTYPES * HIDDEN,), jnp.float32)] * 3
      + [pltpu.VMEM((NCOMBO * HIDDEN,), jnp.float32)]
      + [pltpu.VMEM((chunk + 16,), jnp.int32)] * NBUF
      + [pltpu.VMEM((chunk, HIDDEN), jnp.float32)] * NBUF
      + [pltpu.SemaphoreType.DMA] * (2 * NBUF)
  )
  return pl.kernel(
      body,
      out_type=jax.ShapeDtypeStruct((num_bonds, HIDDEN), jnp.float32),
      mesh=mesh,
      compiler_params=pltpu.CompilerParams(needs_layout_passes=False),
      scratch_types=scratch,
  )


@jax.jit
def kernel(bond_attr, W0, W1, W2):
  n = bond_attr.shape[0]
  a = bond_attr.astype(jnp.int32)
  packed = (a[:, 0] << 8) | (a[:, 1] << 4) | a[:, 2]
  fn = _build(n, 80)
  return fn(packed, W0.reshape(-1), W1.reshape(-1), W2.reshape(-1))


# speedup vs baseline: 3.4316x; 3.4316x over previous
import functools

import jax
import jax.numpy as jnp
from jax import lax
from jax.experimental import pallas as pl
from jax.experimental.pallas import tpu as pltpu
from jax.experimental.pallas import tpu_sc as plsc

HIDDEN = 128
NTYPES = 6
NCOMBO = NTYPES ** 3
NC, NS = 2, 16
NW = NC * NS
NBUF = 5


def _build(num_bonds: int, chunk: int):
  bonds_per_w = num_bonds // NW
  assert bonds_per_w * NW == num_bonds
  nchunk = bonds_per_w // chunk
  assert nchunk * chunk == bonds_per_w and chunk % 16 == 0
  assert nchunk % NBUF == 0 and nchunk >= 2 * NBUF
  nlap = nchunk // NBUF

  def body(attr_hbm, w0_hbm, w1_hbm, w2_hbm, out_hbm, *scratch):
    w0_v, w1_v, w2_v, table_v = scratch[:4]
    attr_bufs = scratch[4:4 + NBUF]
    stage_bufs = scratch[4 + NBUF:4 + 2 * NBUF]
    asems = scratch[4 + 2 * NBUF:4 + 3 * NBUF]
    osems = scratch[4 + 3 * NBUF:4 + 4 * NBUF]

    wid = lax.axis_index("s") * NC + lax.axis_index("c")
    base_bond = wid * bonds_per_w

    for s in range(NBUF):
      pltpu.async_copy(
          attr_hbm.at[pl.ds(base_bond + s * chunk, chunk)],
          attr_bufs[s].at[pl.ds(0, chunk)], asems[s])

    pltpu.sync_copy(w0_hbm, w0_v)
    pltpu.sync_copy(w1_hbm, w1_v)
    pltpu.sync_copy(w2_hbm, w2_v)

    def bi0(i0, c0):
      def bi1(i1, c1):
        def bi2(i2, c2):
          base = (i0 * 36 + i1 * 6 + i2) * HIDDEN
          for k in range(HIDDEN // 16):
            o = k * 16
            table_v[pl.ds(base + o, 16)] = (
                w0_v[pl.ds(i0 * HIDDEN + o, 16)]
                + w1_v[pl.ds(i1 * HIDDEN + o, 16)]
                + w2_v[pl.ds(i2 * HIDDEN + o, 16)])
          return c2
        return lax.fori_loop(0, NTYPES, bi2, c1)
      return lax.fori_loop(0, NTYPES, bi1, c0)
    lax.fori_loop(0, NTYPES, bi0, 0)

    def lap_body(lap, carry):
      for s in range(NBUF):
        t = lap * NBUF + s
        cb = base_bond + t * chunk
        attr_v = attr_bufs[s]
        stage_v = stage_bufs[s]

        pltpu.make_async_copy(
            attr_hbm.at[pl.ds(0, chunk)],
            attr_v.at[pl.ds(0, chunk)], asems[s]).wait()

        @pl.when(lap > 0)
        def _():
          pltpu.make_async_copy(
              stage_v, out_hbm.at[pl.ds(0, chunk), :], osems[s]).wait()

        @plsc.parallel_loop(0, chunk, unroll=4)
        def bond_body(b):
          av = attr_v[pl.ds(b, 16)]
          pk = av[0]
          a0 = pk >> 8
          a1 = (pk >> 4) & 15
          a2 = pk & 15
          base = (a0 * 36 + a1 * NTYPES + a2) * HIDDEN
          for k in range(HIDDEN // 16):
            o = k * 16
            stage_v[b, pl.ds(o, 16)] = table_v[pl.ds(base + o, 16)]

        pltpu.async_copy(stage_v, out_hbm.at[pl.ds(cb, chunk), :], osems[s])

        @pl.when(lap < nlap - 1)
        def _():
          pltpu.async_copy(
              attr_hbm.at[pl.ds(cb + NBUF * chunk, chunk)],
              attr_v.at[pl.ds(0, chunk)], asems[s])
      return carry
    lax.fori_loop(0, nlap, lap_body, 0)

    for s in range(NBUF):
      pltpu.make_async_copy(
          stage_bufs[s], out_hbm.at[pl.ds(0, chunk), :], osems[s]).wait()

  mesh = plsc.VectorSubcoreMesh(
      core_axis_name="c", subcore_axis_name="s", num_cores=NC, num_subcores=NS)
  scratch = (
      [pltpu.VMEM((NTYPES * HIDDEN,), jnp.float32)] * 3
      + [pltpu.VMEM((NCOMBO * HIDDEN,), jnp.float32)]
      + [pltpu.VMEM((chunk + 16,), jnp.int32)] * NBUF
      + [pltpu.VMEM((chunk, HIDDEN), jnp.float32)] * NBUF
      + [pltpu.SemaphoreType.DMA] * (2 * NBUF)
  )
  return pl.kernel(
      body,
      out_type=jax.ShapeDtypeStruct((num_bonds, HIDDEN), jnp.float32),
      mesh=mesh,
      compiler_params=pltpu.CompilerParams(needs_layout_passes=False),
      scratch_types=scratch,
  )


@jax.jit
def kernel(bond_attr, W0, W1, W2):
  n = bond_attr.shape[0]
  a = bond_attr.astype(jnp.int32)
  packed = (a[:, 0] << 8) | (a[:, 1] << 4) | a[:, 2]
  fn = _build(n, 80)
  return fn(packed, W0.reshape(-1), W1.reshape(-1), W2.reshape(-1))


# speedup vs baseline: 3.5153x; 1.0244x over previous
import functools

import jax
import jax.numpy as jnp
from jax import lax
from jax.experimental import pallas as pl
from jax.experimental.pallas import tpu as pltpu
from jax.experimental.pallas import tpu_sc as plsc

HIDDEN = 128
NTYPES = 6
NCOMBO = NTYPES ** 3
NC, NS = 2, 16
NW = NC * NS
NBUF = 5


def _build(num_bonds: int, chunk: int):
  bonds_per_w = num_bonds // NW
  assert bonds_per_w * NW == num_bonds
  nchunk = bonds_per_w // chunk
  assert nchunk * chunk == bonds_per_w and chunk % 16 == 0
  assert nchunk % NBUF == 0 and nchunk >= 2 * NBUF
  nlap = nchunk // NBUF

  def body(attr_hbm, w0_hbm, w1_hbm, w2_hbm, out_hbm, *scratch):
    table_v = scratch[0]
    attr_all = scratch[1]
    stage_bufs = scratch[2:2 + NBUF]
    asems = scratch[2 + NBUF:2 + 2 * NBUF]
    osems = scratch[2 + 2 * NBUF:2 + 3 * NBUF]
    w_v = stage_bufs[0]

    wid = lax.axis_index("s") * NC + lax.axis_index("c")
    base_bond = wid * bonds_per_w

    for s in range(NBUF):
      pltpu.async_copy(
          attr_hbm.at[pl.ds(base_bond + s * chunk, chunk)],
          attr_all.at[pl.ds(s * chunk, chunk)], asems[s])

    pltpu.sync_copy(w0_hbm, w_v.at[pl.ds(0, NTYPES), :])
    pltpu.sync_copy(w1_hbm, w_v.at[pl.ds(NTYPES, NTYPES), :])
    pltpu.sync_copy(w2_hbm, w_v.at[pl.ds(2 * NTYPES, NTYPES), :])

    def bi0(i0, c0):
      def bi1(i1, c1):
        def bi2(i2, c2):
          base = (i0 * 36 + i1 * 6 + i2) * HIDDEN
          for k in range(HIDDEN // 16):
            o = k * 16
            table_v[pl.ds(base + o, 16)] = (
                w_v[i0, pl.ds(o, 16)]
                + w_v[NTYPES + i1, pl.ds(o, 16)]
                + w_v[2 * NTYPES + i2, pl.ds(o, 16)])
          return c2
        return lax.fori_loop(0, NTYPES, bi2, c1)
      return lax.fori_loop(0, NTYPES, bi1, c0)
    lax.fori_loop(0, NTYPES, bi0, 0)

    def lap_body(lap, carry):
      for s in range(NBUF):
        t = lap * NBUF + s
        cb = base_bond + t * chunk
        stage_v = stage_bufs[s]

        pltpu.make_async_copy(
            attr_hbm.at[pl.ds(0, chunk)],
            attr_all.at[pl.ds(s * chunk, chunk)], asems[s]).wait()

        @pl.when(lap > 0)
        def _():
          pltpu.make_async_copy(
              stage_v, out_hbm.at[pl.ds(0, chunk), :], osems[s]).wait()

        @plsc.parallel_loop(0, chunk, unroll=4)
        def bond_body(b):
          av = attr_all[pl.ds(s * chunk + b, 16)]
          pk = av[0]
          a0 = pk >> 8
          a1 = (pk >> 4) & 15
          a2 = pk & 15
          base = (a0 * 36 + a1 * NTYPES + a2) * HIDDEN
          for k in range(HIDDEN // 16):
            o = k * 16
            stage_v[b, pl.ds(o, 16)] = table_v[pl.ds(base + o, 16)]

        pltpu.async_copy(stage_v, out_hbm.at[pl.ds(cb, chunk), :], osems[s])

        @pl.when(lap < nlap - 1)
        def _():
          pltpu.async_copy(
              attr_hbm.at[pl.ds(cb + NBUF * chunk, chunk)],
              attr_all.at[pl.ds(s * chunk, chunk)], asems[s])
      return carry
    lax.fori_loop(0, nlap, lap_body, 0)

    for s in range(NBUF):
      pltpu.make_async_copy(
          stage_bufs[s], out_hbm.at[pl.ds(0, chunk), :], osems[s]).wait()

  mesh = plsc.VectorSubcoreMesh(
      core_axis_name="c", subcore_axis_name="s", num_cores=NC, num_subcores=NS)
  scratch = (
      [pltpu.VMEM((NCOMBO * HIDDEN,), jnp.float32)]
      + [pltpu.VMEM((NBUF * chunk + 16,), jnp.int32)]
      + [pltpu.VMEM((chunk, HIDDEN), jnp.float32)] * NBUF
      + [pltpu.SemaphoreType.DMA] * (2 * NBUF)
  )
  return pl.kernel(
      body,
      out_type=jax.ShapeDtypeStruct((num_bonds, HIDDEN), jnp.float32),
      mesh=mesh,
      compiler_params=pltpu.CompilerParams(needs_layout_passes=False),
      scratch_types=scratch,
  )


@jax.jit
def kernel(bond_attr, W0, W1, W2):
  n = bond_attr.shape[0]
  a = bond_attr.astype(jnp.int32)
  packed = (a[:, 0] << 8) | (a[:, 1] << 4) | a[:, 2]
  fn = _build(n, 160)
  return fn(packed, W0, W1, W2)


# speedup vs baseline: 3.5166x; 1.0004x over previous
import jax
import jax.numpy as jnp
from jax import lax
from jax.experimental import pallas as pl
from jax.experimental.pallas import tpu as pltpu
from jax.experimental.pallas import tpu_sc as plsc

HIDDEN = 128
NTYPES = 6
NCOMBO = NTYPES ** 3
NC, NS = 2, 16
NW = NC * NS
NBUF = 5


def _build(num_bonds: int, chunk: int):
  bonds_per_w = num_bonds // NW
  assert bonds_per_w * NW == num_bonds
  nchunk = bonds_per_w // chunk
  assert nchunk * chunk == bonds_per_w and chunk % 16 == 0
  assert nchunk % NBUF == 0 and nchunk >= 2 * NBUF
  nlap = nchunk // NBUF

  def body(attr_hbm, w0_hbm, w1_hbm, w2_hbm, out_hbm, *scratch):
    table_v = scratch[0]
    attr_all = scratch[1]
    stage_bufs = scratch[2:2 + NBUF]
    asems = scratch[2 + NBUF:2 + 2 * NBUF]
    osems = scratch[2 + 2 * NBUF:2 + 3 * NBUF]
    w_v = stage_bufs[0]

    wid = lax.axis_index("s") * NC + lax.axis_index("c")
    base_bond = wid * bonds_per_w

    for s in range(NBUF):
      pltpu.async_copy(
          attr_hbm.at[pl.ds(base_bond + s * chunk, chunk)],
          attr_all.at[pl.ds(s * chunk, chunk)], asems[s])

    pltpu.sync_copy(w0_hbm, w_v.at[pl.ds(0, NTYPES), :])
    pltpu.sync_copy(w1_hbm, w_v.at[pl.ds(NTYPES, NTYPES), :])
    pltpu.sync_copy(w2_hbm, w_v.at[pl.ds(2 * NTYPES, NTYPES), :])

    def bi0(i0, c0):
      def bi1(i1, c1):
        def bi2(i2, c2):
          base = (i0 * 36 + i1 * 6 + i2) * HIDDEN
          for k in range(HIDDEN // 16):
            o = k * 16
            table_v[pl.ds(base + o, 16)] = (
                w_v[i0, pl.ds(o, 16)]
                + w_v[NTYPES + i1, pl.ds(o, 16)]
                + w_v[2 * NTYPES + i2, pl.ds(o, 16)])
          return c2
        return lax.fori_loop(0, NTYPES, bi2, c1)
      return lax.fori_loop(0, NTYPES, bi1, c0)
    lax.fori_loop(0, NTYPES, bi0, 0)

    def lap_body(lap, carry):
      for s in range(NBUF):
        t = lap * NBUF + s
        cb = base_bond + t * chunk
        stage_v = stage_bufs[s]

        pltpu.make_async_copy(
            attr_hbm.at[pl.ds(0, chunk)],
            attr_all.at[pl.ds(s * chunk, chunk)], asems[s]).wait()

        @pl.when(lap > 0)
        def _():
          pltpu.make_async_copy(
              stage_v, out_hbm.at[pl.ds(0, chunk), :], osems[s]).wait()

        @plsc.parallel_loop(0, chunk, unroll=4)
        def bond_body(b):
          av = attr_all[pl.ds(s * chunk + b, 16)]
          pk = av[0]
          a0 = pk >> 8
          a1 = (pk >> 4) & 15
          a2 = pk & 15
          base = (a0 * 36 + a1 * NTYPES + a2) * HIDDEN
          for k in range(HIDDEN // 16):
            o = k * 16
            stage_v[b, pl.ds(o, 16)] = table_v[pl.ds(base + o, 16)]

        pltpu.async_copy(stage_v, out_hbm.at[pl.ds(cb, chunk), :], osems[s])

        @pl.when(lap < nlap - 1)
        def _():
          pltpu.async_copy(
              attr_hbm.at[pl.ds(cb + NBUF * chunk, chunk)],
              attr_all.at[pl.ds(s * chunk, chunk)], asems[s])
      return carry
    lax.fori_loop(0, nlap, lap_body, 0)

    for s in range(NBUF):
      pltpu.make_async_copy(
          stage_bufs[s], out_hbm.at[pl.ds(0, chunk), :], osems[s]).wait()

  mesh = plsc.VectorSubcoreMesh(
      core_axis_name="c", subcore_axis_name="s", num_cores=NC, num_subcores=NS)
  scratch = (
      [pltpu.VMEM((NCOMBO * HIDDEN,), jnp.float32)]
      + [pltpu.VMEM((NBUF * chunk + 16,), jnp.int32)]
      + [pltpu.VMEM((chunk, HIDDEN), jnp.float32)] * NBUF
      + [pltpu.SemaphoreType.DMA] * (2 * NBUF)
  )
  return pl.kernel(
      body,
      out_type=jax.ShapeDtypeStruct((num_bonds, HIDDEN), jnp.float32),
      mesh=mesh,
      compiler_params=pltpu.CompilerParams(needs_layout_passes=False),
      scratch_types=scratch,
  )


@jax.jit
def kernel(bond_attr, W0, W1, W2):
  n = bond_attr.shape[0]
  a = bond_attr.astype(jnp.int32)
  packed = (a[:, 0] << 8) | (a[:, 1] << 4) | a[:, 2]
  fn = _build(n, 160)
  return fn(packed, W0, W1, W2)
